# baseline (speedup 1.0000x reference)
import functools
import math

import jax
import jax.numpy as jnp
from jax import lax
from jax.experimental import pallas as pl
from jax.experimental.pallas import tpu as pltpu
from jax.experimental.pallas import tpu_sc as plsc

N_NODES = 100000
N_EDGES = 6400000
CUTOFF = 5.0
N_BASES = 8

NC = 2
NS = 16
NW = NC * NS
PER_TILE = N_EDGES // NW
B = 2000
NCHUNKS = PER_TILE // B
GROUPS = B // 16
N_PAD = 100352
FILL = N_PAD // NS

BE = 1024
GRID = N_EDGES // BE

_PI = float(math.pi)
_PREF = float(math.sqrt(2.0 / CUTOFF))


def _sc_body(px_hbm, py_hbm, pz_hbm, send_hbm, recv_hbm, shifts_hbm,
             len_hbm, ux_hbm, uy_hbm, uz_hbm,
             sidx, ridx, sx_b, sy_b, sz_b, rx_b, ry_b, rz_b,
             shv, lenb, uxb, uyb, uzb, px_sh, py_sh, pz_sh, gsem):
    w = lax.axis_index("s") * NC + lax.axis_index("c")
    iota = lax.iota(jnp.int32, 16)

    sid = lax.axis_index("s")
    fsl = pl.ds(sid * FILL, FILL)
    pltpu.sync_copy(px_hbm.at[fsl], px_sh.at[fsl])
    pltpu.sync_copy(py_hbm.at[fsl], py_sh.at[fsl])
    pltpu.sync_copy(pz_hbm.at[fsl], pz_sh.at[fsl])
    plsc.subcore_barrier()

    def chunk_body(ci, carry):
        base = w * PER_TILE + ci * B
        pltpu.sync_copy(send_hbm.at[pl.ds(base, B)], sidx)
        pltpu.sync_copy(recv_hbm.at[pl.ds(base, B)], ridx)
        pltpu.sync_copy(shifts_hbm.at[pl.ds(3 * base, 3 * B)], shv)

        descs = [
            pltpu.async_copy(px_sh.at[sidx], sx_b, gsem),
            pltpu.async_copy(py_sh.at[sidx], sy_b, gsem),
            pltpu.async_copy(pz_sh.at[sidx], sz_b, gsem),
            pltpu.async_copy(px_sh.at[ridx], rx_b, gsem),
            pltpu.async_copy(py_sh.at[ridx], ry_b, gsem),
            pltpu.async_copy(pz_sh.at[ridx], rz_b, gsem),
        ]
        for d in descs:
            d.wait()

        def grp(g, c2):
            o = g * 16
            sl16 = pl.ds(o, 16)
            rows3 = (iota + o) * 3
            vx = rx_b[sl16] - sx_b[sl16] + plsc.load_gather(shv, [rows3])
            vy = ry_b[sl16] - sy_b[sl16] + plsc.load_gather(shv, [rows3 + 1])
            vz = rz_b[sl16] - sz_b[sl16] + plsc.load_gather(shv, [rows3 + 2])
            len2 = vx * vx + vy * vy + vz * vz
            bits = plsc.bitcast(len2, jnp.int32)
            y = plsc.bitcast(jnp.int32(0x5F3759DF) - (bits >> 1), jnp.float32)
            h = len2 * 0.5
            y = y * (1.5 - h * y * y)
            y = y * (1.5 - h * y * y)
            y = y * (1.5 - h * y * y)
            lenb[sl16] = len2 * y
            uxb[sl16] = vx * y
            uyb[sl16] = vy * y
            uzb[sl16] = vz * y
            return c2

        lax.fori_loop(0, GROUPS, grp, 0, unroll=False)

        pltpu.sync_copy(lenb, len_hbm.at[pl.ds(base, B)])
        pltpu.sync_copy(uxb, ux_hbm.at[pl.ds(base, B)])
        pltpu.sync_copy(uyb, uy_hbm.at[pl.ds(base, B)])
        pltpu.sync_copy(uzb, uz_hbm.at[pl.ds(base, B)])
        return carry

    lax.fori_loop(0, NCHUNKS, chunk_body, 0, unroll=False)


def _tc_body(len_ref, ux_ref, uy_ref, uz_ref, emb_ref, unit_ref):
    l = len_ref[...]
    n = (lax.broadcasted_iota(jnp.int32, (BE, N_BASES), 1) + 1
         ).astype(jnp.float32)
    a = (l * (_PI / CUTOFF)) * n
    r5 = l * (1.0 / CUTOFF)
    rr2 = r5 * r5
    r6 = rr2 * r5
    r6 = r6 * r6
    env = 1.0 + r6 * (-28.0 + 48.0 * r5 - 21.0 * rr2)
    env = jnp.where(l < CUTOFF, env, 0.0)
    emb_ref[...] = jnp.sin(a) * ((_PREF * env) / l)
    unit_ref[...] = jnp.concatenate(
        [ux_ref[...], uy_ref[...], uz_ref[...]], axis=1)


@jax.jit
def kernel(positions, edge_index, shifts):
    ppad = jnp.pad(positions, ((0, N_PAD - N_NODES), (0, 0)))
    px = ppad[:, 0]
    py = ppad[:, 1]
    pz = ppad[:, 2]
    sender = edge_index[0]
    receiver = edge_index[1]
    shifts_flat = shifts.reshape(3 * N_EDGES)

    mesh = plsc.VectorSubcoreMesh(core_axis_name="c", subcore_axis_name="s")
    sc_fn = functools.partial(
        pl.kernel,
        mesh=mesh,
        compiler_params=pltpu.CompilerParams(needs_layout_passes=False),
        out_type=[
            jax.ShapeDtypeStruct((N_EDGES,), jnp.float32),
            jax.ShapeDtypeStruct((N_EDGES,), jnp.float32),
            jax.ShapeDtypeStruct((N_EDGES,), jnp.float32),
            jax.ShapeDtypeStruct((N_EDGES,), jnp.float32),
        ],
        scratch_types=[
            pltpu.VMEM((B,), jnp.int32),
            pltpu.VMEM((B,), jnp.int32),
            pltpu.VMEM((B,), jnp.float32),
            pltpu.VMEM((B,), jnp.float32),
            pltpu.VMEM((B,), jnp.float32),
            pltpu.VMEM((B,), jnp.float32),
            pltpu.VMEM((B,), jnp.float32),
            pltpu.VMEM((B,), jnp.float32),
            pltpu.VMEM((3 * B,), jnp.float32),
            pltpu.VMEM((B,), jnp.float32),
            pltpu.VMEM((B,), jnp.float32),
            pltpu.VMEM((B,), jnp.float32),
            pltpu.VMEM((B,), jnp.float32),
            pltpu.VMEM_SHARED((N_PAD,), jnp.float32),
            pltpu.VMEM_SHARED((N_PAD,), jnp.float32),
            pltpu.VMEM_SHARED((N_PAD,), jnp.float32),
            pltpu.SemaphoreType.DMA,
        ],
    )(_sc_body)
    lengths, ux, uy, uz = sc_fn(px, py, pz, sender, receiver, shifts_flat)

    lengths2d = lengths.reshape(N_EDGES, 1)
    emb, unit = pl.pallas_call(
        _tc_body,
        grid=(GRID,),
        in_specs=[
            pl.BlockSpec((BE, 1), lambda i: (i, 0)),
            pl.BlockSpec((BE, 1), lambda i: (i, 0)),
            pl.BlockSpec((BE, 1), lambda i: (i, 0)),
            pl.BlockSpec((BE, 1), lambda i: (i, 0)),
        ],
        out_specs=[
            pl.BlockSpec((BE, N_BASES), lambda i: (i, 0)),
            pl.BlockSpec((BE, 3), lambda i: (i, 0)),
        ],
        out_shape=[
            jax.ShapeDtypeStruct((N_EDGES, N_BASES), jnp.float32),
            jax.ShapeDtypeStruct((N_EDGES, 3), jnp.float32),
        ],
    )(lengths2d, ux.reshape(N_EDGES, 1), uy.reshape(N_EDGES, 1),
      uz.reshape(N_EDGES, 1))

    return (lengths2d, emb, unit)


# speedup vs baseline: 2.3729x; 2.3729x over previous
import functools
import math

import jax
import jax.numpy as jnp
from jax import lax
from jax.experimental import pallas as pl
from jax.experimental.pallas import tpu as pltpu
from jax.experimental.pallas import tpu_sc as plsc

N_NODES = 100000
N_EDGES = 6400000
CUTOFF = 5.0
N_BASES = 8

NC = 2
NS = 16
NW = NC * NS
PER_TILE = N_EDGES // NW
B = 2000
NCHUNKS = PER_TILE // B
GROUPS = B // 16
N_PAD = 100352
FILL = N_PAD // NS

_PI = float(math.pi)
_HALF_PI = float(math.pi / 2.0)
_PREF = float(math.sqrt(2.0 / CUTOFF))


TC_BR = 512


def _tc_emb(len_ref, emb_ref):
    l = len_ref[...]
    r5 = l * (1.0 / CUTOFF)
    rr2 = r5 * r5
    r6 = rr2 * r5
    r6 = r6 * r6
    env = 1.0 + r6 * (-28.0 + 48.0 * r5 - 21.0 * rr2)
    env = jnp.where(l < CUTOFF, env, 0.0)
    k = (_PREF * env) / l
    a = l * (_PI / CUTOFF)
    for n in range(N_BASES):
        emb_ref[:, pl.ds(128 * n, 128)] = jnp.sin(a * float(n + 1)) * k


def _tile_body(px_hbm, py_hbm, pz_hbm, send_hbm, recv_hbm, shifts_hbm,
               len_hbm, unit_hbm,
               sidx, ridx, sx_b, sy_b, sz_b, rx_b, ry_b, rz_b,
               shv, lenb, unitb, px_sh, py_sh, pz_sh, gsem):
    w = lax.axis_index("s") * NC + lax.axis_index("c")
    iota = lax.iota(jnp.int32, 16)

    sid = lax.axis_index("s")
    foff = sid * FILL
    fsl = pl.ds(foff, FILL)
    pltpu.sync_copy(px_hbm.at[fsl], px_sh.at[fsl])
    pltpu.sync_copy(py_hbm.at[fsl], py_sh.at[fsl])
    pltpu.sync_copy(pz_hbm.at[fsl], pz_sh.at[fsl])
    plsc.subcore_barrier()

    def chunk_body(ci, carry):
        base = w * PER_TILE + ci * B
        pltpu.sync_copy(send_hbm.at[pl.ds(base, B)], sidx)
        pltpu.sync_copy(recv_hbm.at[pl.ds(base, B)], ridx)
        pltpu.sync_copy(shifts_hbm.at[pl.ds(3 * base, 3 * B)], shv)

        descs = [
            pltpu.async_copy(px_sh.at[sidx], sx_b, gsem),
            pltpu.async_copy(py_sh.at[sidx], sy_b, gsem),
            pltpu.async_copy(pz_sh.at[sidx], sz_b, gsem),
            pltpu.async_copy(px_sh.at[ridx], rx_b, gsem),
            pltpu.async_copy(py_sh.at[ridx], ry_b, gsem),
            pltpu.async_copy(pz_sh.at[ridx], rz_b, gsem),
        ]
        for d in descs:
            d.wait()

        def grp(g, c2):
            o = g * 16
            sl16 = pl.ds(o, 16)
            rows3 = (iota + o) * 3
            vx = rx_b[sl16] - sx_b[sl16] + plsc.load_gather(shv, [rows3])
            vy = ry_b[sl16] - sy_b[sl16] + plsc.load_gather(shv, [rows3 + 1])
            vz = rz_b[sl16] - sz_b[sl16] + plsc.load_gather(shv, [rows3 + 2])
            len2 = vx * vx + vy * vy + vz * vz
            bits = plsc.bitcast(len2, jnp.int32)
            y = plsc.bitcast(jnp.int32(0x5F3759DF) - (bits >> 1), jnp.float32)
            h = len2 * 0.5
            y = y * (1.5 - h * y * y)
            y = y * (1.5 - h * y * y)
            y = y * (1.5 - h * y * y)
            ln = len2 * y
            lenb[sl16] = ln
            plsc.store_scatter(unitb, [rows3], vx * y)
            plsc.store_scatter(unitb, [rows3 + 1], vy * y)
            plsc.store_scatter(unitb, [rows3 + 2], vz * y)
            return c2

        lax.fori_loop(0, GROUPS, grp, 0, unroll=False)

        pltpu.sync_copy(lenb, len_hbm.at[pl.ds(base, B)])
        pltpu.sync_copy(unitb, unit_hbm.at[pl.ds(3 * base, 3 * B)])
        return carry

    lax.fori_loop(0, NCHUNKS, chunk_body, 0, unroll=False)


@jax.jit
def kernel(positions, edge_index, shifts):
    ppad = jnp.pad(positions, ((0, N_PAD - N_NODES), (0, 0)))
    px = ppad[:, 0]
    py = ppad[:, 1]
    pz = ppad[:, 2]
    sender = edge_index[0]
    receiver = edge_index[1]
    shifts_flat = shifts.reshape(3 * N_EDGES)

    mesh = plsc.VectorSubcoreMesh(core_axis_name="c", subcore_axis_name="s")
    fn = functools.partial(
        pl.kernel,
        mesh=mesh,
        compiler_params=pltpu.CompilerParams(needs_layout_passes=False),
        out_type=[
            jax.ShapeDtypeStruct((N_EDGES,), jnp.float32),
            jax.ShapeDtypeStruct((N_EDGES * 3,), jnp.float32),
        ],
        scratch_types=[
            pltpu.VMEM((B,), jnp.int32),
            pltpu.VMEM((B,), jnp.int32),
            pltpu.VMEM((B,), jnp.float32),
            pltpu.VMEM((B,), jnp.float32),
            pltpu.VMEM((B,), jnp.float32),
            pltpu.VMEM((B,), jnp.float32),
            pltpu.VMEM((B,), jnp.float32),
            pltpu.VMEM((B,), jnp.float32),
            pltpu.VMEM((3 * B,), jnp.float32),
            pltpu.VMEM((B,), jnp.float32),
            pltpu.VMEM((3 * B,), jnp.float32),
            pltpu.VMEM_SHARED((N_PAD,), jnp.float32),
            pltpu.VMEM_SHARED((N_PAD,), jnp.float32),
            pltpu.VMEM_SHARED((N_PAD,), jnp.float32),
            pltpu.SemaphoreType.DMA,
        ],
    )(_tile_body)
    lengths, unit_flat = fn(px, py, pz, sender, receiver, shifts_flat)

    len_mat = lengths.reshape(N_EDGES // 128, 128)
    emb_mat = pl.pallas_call(
        _tc_emb,
        grid=(N_EDGES // 128 // TC_BR,),
        in_specs=[pl.BlockSpec((TC_BR, 128), lambda i: (i, 0))],
        out_specs=pl.BlockSpec((TC_BR, 128 * N_BASES), lambda i: (i, 0)),
        out_shape=jax.ShapeDtypeStruct(
            (N_EDGES // 128, 128 * N_BASES), jnp.float32),
    )(len_mat)
    emb = (emb_mat.reshape(N_EDGES // 128, N_BASES, 128)
           .swapaxes(1, 2).reshape(N_EDGES, N_BASES))
    return (lengths.reshape(N_EDGES, 1), emb,
            unit_flat.reshape(N_EDGES, 3))
